# Initial kernel scaffold; baseline (speedup 1.0000x reference)
import jax
import jax.numpy as jnp
from jax.experimental import pallas as pl


def kernel(query, query_len, document, table, W1, b1, W2, b2, W3, b3, Wg, bg):
    raise NotImplementedError("write your pallas kernel here")


# speedup vs baseline: 6.5500x; 6.5500x over previous
import functools

import jax
import jax.numpy as jnp
from jax import lax
from jax.experimental import pallas as pl
from jax.experimental.pallas import tpu as pltpu
from jax.experimental.pallas import tpu_sc as plsc

BATCH = 1024
LQ = 20
LD = 200
EMB = 128
NBINS = 30
BB = 8
CH = 128
DEPTH = 5


def _bin_thresholds():
    import numpy as np
    w = np.float32(2.0 / NBINS)
    out = []
    for k in range(1, NBINS):
        y = np.float32(np.float64(k) * np.float64(w))
        kf = np.float32(k)
        if np.float32(y / w) >= kf:
            while True:
                y2 = np.nextafter(y, np.float32(-np.inf), dtype=np.float32)
                if np.float32(y2 / w) >= kf:
                    y = y2
                else:
                    break
        else:
            while np.float32(y / w) < kf:
                y = np.nextafter(y, np.float32(np.inf), dtype=np.float32)
        out.append(float(y))
    return out


_THRESH = _bin_thresholds()


def _refined_rsqrt(x):
    x = jnp.maximum(x, jnp.float32(1e-16))
    r = lax.rsqrt(x)
    return r * (jnp.float32(1.5) - jnp.float32(0.5) * x * r * r)


def _b16(x):
    return x.astype(jnp.bfloat16).astype(jnp.float32)


def _sc_gather(table, qidx, didx):
    info = plsc.get_sparse_core_info()
    nc, ns = info.num_cores, info.num_subcores
    nw = nc * ns
    nq, nd = BATCH * LQ, BATCH * LD
    nch_q = nq // (nw * CH)
    nch_d = nd // (nw * CH)
    assert nq % (nw * CH) == 0 and nd % (nw * CH) == 0
    assert nch_q % DEPTH == 0 and nch_d % DEPTH == 0

    qidx3 = qidx.reshape(nw, nch_q, CH)
    didx3 = didx.reshape(nw, nch_d, CH)

    mesh = plsc.VectorSubcoreMesh(core_axis_name="c", subcore_axis_name="s")

    @functools.partial(
        pl.kernel,
        mesh=mesh,
        out_type=[
            jax.ShapeDtypeStruct((nq, EMB), jnp.float32),
            jax.ShapeDtypeStruct((nd, EMB), jnp.float32),
        ],
        scratch_types=[
            pltpu.VMEM((nch_q, CH), jnp.int32),
            pltpu.VMEM((nch_d, CH), jnp.int32),
            pltpu.VMEM((DEPTH, CH, EMB), jnp.float32),
        ]
        + [pltpu.SemaphoreType.DMA] * (2 * DEPTH),
    )
    def gather_kernel(tbl, qi3, di3, qout, dout, qidx_v, didx_v, rows_v, *sems):
        gsem = sems[:DEPTH]
        osem = sems[DEPTH:]
        wid = lax.axis_index("s") * nc + lax.axis_index("c")
        pltpu.sync_copy(di3.at[wid], didx_v)
        pltpu.sync_copy(qi3.at[wid], qidx_v)

        def phase(idx_v, out_hbm, nch):
            base = wid * (nch * CH)

            def round_body(r, carry):
                for b in range(DEPTH):
                    c = r * DEPTH + b
                    pltpu.make_async_copy(
                        tbl.at[idx_v.at[c]], rows_v.at[b], gsem[b]
                    ).start()
                for b in range(DEPTH):
                    c = r * DEPTH + b
                    pltpu.make_async_copy(
                        tbl.at[idx_v.at[c]], rows_v.at[b], gsem[b]
                    ).wait()
                    pltpu.make_async_copy(
                        rows_v.at[b],
                        out_hbm.at[pl.ds(base + c * CH, CH)],
                        osem[b],
                    ).start()
                for b in range(DEPTH):
                    c = r * DEPTH + b
                    pltpu.make_async_copy(
                        rows_v.at[b],
                        out_hbm.at[pl.ds(base + c * CH, CH)],
                        osem[b],
                    ).wait()
                return carry

            lax.fori_loop(0, nch // DEPTH, round_body, 0)

        phase(didx_v, dout, nch_d)
        phase(qidx_v, qout, nch_q)

    return gather_kernel(table, qidx3, didx3)


def _tc_body(qe_ref, de_ref, qlen_ref, w1t_ref, b1_ref, w2t_ref, b2_ref,
             w3_ref, b3_ref, wgt_ref, bg_ref, out_ref):
    col0 = (lax.broadcasted_iota(jnp.int32, (1, 8), 1) == 0).astype(jnp.float32)
    iota_bb = lax.broadcasted_iota(jnp.int32, (1, BB), 1)
    qrow = lax.broadcasted_iota(jnp.int32, (LQ, 1), 0).astype(jnp.float32)
    nt = (((1,), (1,)), ((), ()))
    nn = (((1,), (0,)), ((), ()))

    ones_row = jnp.ones((1, EMB), jnp.float32)
    row_acc = jnp.zeros((1, BB), jnp.float32)
    for i in range(BB):
        q = qe_ref[i]
        d = de_ref[i]
        dots = lax.dot_general(q.astype(jnp.bfloat16), d.astype(jnp.bfloat16),
                               nt, preferred_element_type=jnp.float32)
        rq = _refined_rsqrt(jnp.sum(q * q, axis=1, keepdims=True))
        dsq = d * d
        dsq_h = _b16(dsq)
        dsq_l = dsq - dsq_h
        dn2 = (lax.dot_general(ones_row, dsq_h, nt,
                               preferred_element_type=jnp.float32)
               + lax.dot_general(ones_row, dsq_l, nt,
                                 preferred_element_type=jnp.float32))
        inter = dots * rq * _refined_rsqrt(dn2)
        y = inter + jnp.float32(1.0)
        s = [None] * (NBINS + 1)
        for k in range(1, NBINS):
            s[k] = jnp.sum((y >= jnp.float32(_THRESH[k - 1])).astype(jnp.float32),
                           axis=1, keepdims=True)
        mask = (qrow < qlen_ref[i]).astype(jnp.float32)
        acc = jnp.zeros((LQ, 8), jnp.float32)
        for k in range(NBINS):
            if k == 0:
                cnt = jnp.float32(LD) - s[1]
            elif k == NBINS - 1:
                cnt = s[NBINS - 1]
            else:
                cnt = s[k] - s[k + 1]
            lh = _b16(jnp.log1p(cnt * mask))
            acc = acc + lh * w1t_ref[k:k + 1, :]
        z1 = jnp.tanh(acc + b1_ref[...])
        z2 = jnp.tanh(lax.dot_general(z1, w2t_ref[...], nn,
                                      preferred_element_type=jnp.float32)
                      + b2_ref[...])
        z3 = jnp.tanh(z2 * w3_ref[...] + b3_ref[...])
        glog = lax.dot_general(q, wgt_ref[...], nn,
                               preferred_element_type=jnp.float32) + bg_ref[...]
        gm = glog * col0 + (col0 - 1.0) * jnp.float32(1e9)
        m = jnp.max(gm)
        e = jnp.exp(gm - m)
        score = jnp.sum(z3 * e) / jnp.sum(e)
        row_acc = row_acc + jnp.where(iota_bb == i, score, 0.0)
    out_ref[...] = row_acc.reshape(1, 1, BB)


def _tc_call(qe, de, qlenf, w1t, b1p, w2t, b2p, w3p, b3p, wgt, bgp):
    grid = BATCH // BB
    full = lambda shape: pl.BlockSpec(shape, lambda i: (0,) * len(shape))
    out = pl.pallas_call(
        _tc_body,
        grid=(grid,),
        in_specs=[
            pl.BlockSpec((BB, LQ, EMB), lambda i: (i, 0, 0)),
            pl.BlockSpec((BB, LD, EMB), lambda i: (i, 0, 0)),
            pl.BlockSpec((BB, 1, 1), lambda i: (i, 0, 0)),
            full((32, 8)), full((1, 8)), full((8, 8)), full((1, 8)),
            full((1, 8)), full((1, 8)), full((EMB, 8)), full((1, 8)),
        ],
        out_specs=pl.BlockSpec((1, 1, BB), lambda i: (i, 0, 0)),
        out_shape=jax.ShapeDtypeStruct((grid, 1, BB), jnp.float32),
    )(qe, de, qlenf, w1t, b1p, w2t, b2p, w3p, b3p, wgt, bgp)
    return out.reshape(BATCH)


def kernel(query, query_len, document, table, W1, b1, W2, b2, W3, b3, Wg, bg):
    qidx = query.reshape(-1).astype(jnp.int32)
    didx = document.reshape(-1).astype(jnp.int32)
    q_emb, d_emb = _sc_gather(table, qidx, didx)
    qe = q_emb.reshape(BATCH, LQ, EMB)
    de = d_emb.reshape(BATCH, LD, EMB)
    qlenf = query_len.astype(jnp.float32).reshape(BATCH, 1, 1)
    f32 = jnp.float32
    r16 = lambda x: x.astype(jnp.bfloat16).astype(jnp.float32)
    w1t = jnp.zeros((32, 8), f32).at[:NBINS, :5].set(r16(W1.T))
    b1p = jnp.zeros((1, 8), f32).at[0, :5].set(b1)
    w2t = jnp.zeros((8, 8), f32).at[:5, 0].set(W2[0])
    b2p = jnp.zeros((1, 8), f32).at[0, 0].set(b2[0])
    w3p = jnp.zeros((1, 8), f32).at[0, 0].set(W3[0, 0])
    b3p = jnp.zeros((1, 8), f32).at[0, 0].set(b3[0])
    wgt = jnp.zeros((EMB, 8), f32).at[:, 0].set(Wg[0])
    bgp = jnp.zeros((1, 8), f32).at[0, 0].set(bg[0])
    return _tc_call(qe, de, qlenf, w1t, b1p, w2t, b2p, w3p, b3p, wgt, bgp)


# speedup vs baseline: 6.7497x; 1.0305x over previous
import functools

import jax
import jax.numpy as jnp
from jax import lax
from jax.experimental import pallas as pl
from jax.experimental.pallas import tpu as pltpu
from jax.experimental.pallas import tpu_sc as plsc

BATCH = 1024
LQ = 20
LD = 200
EMB = 128
NBINS = 30
BB = 8
CH = 128
DEPTH = 5


def _bin_thresholds():
    import numpy as np
    w = np.float32(2.0 / NBINS)
    out = []
    for k in range(1, NBINS):
        y = np.float32(np.float64(k) * np.float64(w))
        kf = np.float32(k)
        if np.float32(y / w) >= kf:
            while True:
                y2 = np.nextafter(y, np.float32(-np.inf), dtype=np.float32)
                if np.float32(y2 / w) >= kf:
                    y = y2
                else:
                    break
        else:
            while np.float32(y / w) < kf:
                y = np.nextafter(y, np.float32(np.inf), dtype=np.float32)
        out.append(float(y))
    return out


_THRESH = _bin_thresholds()


def _refined_rsqrt(x):
    x = jnp.maximum(x, jnp.float32(1e-16))
    r = lax.rsqrt(x)
    return r * (jnp.float32(1.5) - jnp.float32(0.5) * x * r * r)


def _b16(x):
    return x.astype(jnp.bfloat16).astype(jnp.float32)


def _sc_gather(table, qidx, didx):
    info = plsc.get_sparse_core_info()
    nc, ns = info.num_cores, info.num_subcores
    nw = nc * ns
    nq, nd = BATCH * LQ, BATCH * LD
    nch_q = nq // (nw * CH)
    nch_d = nd // (nw * CH)
    assert nq % (nw * CH) == 0 and nd % (nw * CH) == 0
    assert nch_q % DEPTH == 0 and nch_d % DEPTH == 0

    qidx3 = qidx.reshape(nw, nch_q, CH)
    didx3 = didx.reshape(nw, nch_d, CH)

    mesh = plsc.VectorSubcoreMesh(core_axis_name="c", subcore_axis_name="s")

    @functools.partial(
        pl.kernel,
        mesh=mesh,
        out_type=[
            jax.ShapeDtypeStruct((nq, EMB), jnp.float32),
            jax.ShapeDtypeStruct((nd, EMB), jnp.float32),
        ],
        scratch_types=[
            pltpu.VMEM((nch_q, CH), jnp.int32),
            pltpu.VMEM((nch_d, CH), jnp.int32),
            pltpu.VMEM((DEPTH, CH, EMB), jnp.float32),
        ]
        + [pltpu.SemaphoreType.DMA] * (2 * DEPTH),
    )
    def gather_kernel(tbl, qi3, di3, qout, dout, qidx_v, didx_v, rows_v, *sems):
        gsem = sems[:DEPTH]
        osem = sems[DEPTH:]
        wid = lax.axis_index("s") * nc + lax.axis_index("c")
        pltpu.sync_copy(di3.at[wid], didx_v)
        pltpu.sync_copy(qi3.at[wid], qidx_v)

        def phase(idx_v, out_hbm, nch):
            base = wid * (nch * CH)

            def round_body(r, carry):
                for b in range(DEPTH):
                    c = r * DEPTH + b
                    pltpu.make_async_copy(
                        tbl.at[idx_v.at[c]], rows_v.at[b], gsem[b]
                    ).start()
                for b in range(DEPTH):
                    c = r * DEPTH + b
                    pltpu.make_async_copy(
                        tbl.at[idx_v.at[c]], rows_v.at[b], gsem[b]
                    ).wait()
                    pltpu.make_async_copy(
                        rows_v.at[b],
                        out_hbm.at[pl.ds(base + c * CH, CH)],
                        osem[b],
                    ).start()
                for b in range(DEPTH):
                    c = r * DEPTH + b
                    pltpu.make_async_copy(
                        rows_v.at[b],
                        out_hbm.at[pl.ds(base + c * CH, CH)],
                        osem[b],
                    ).wait()
                return carry

            lax.fori_loop(0, nch // DEPTH, round_body, 0)

        phase(didx_v, dout, nch_d)
        phase(qidx_v, qout, nch_q)

    return gather_kernel(table, qidx3, didx3)


def _tc_body(qe_ref, de_ref, qlen_ref, w1t_ref, b1_ref, w2t_ref, b2_ref,
             w3_ref, b3_ref, wgt_ref, bg_ref, out_ref):
    col0 = (lax.broadcasted_iota(jnp.int32, (1, 8), 1) == 0).astype(jnp.float32)
    iota_bb = lax.broadcasted_iota(jnp.int32, (1, BB), 1)
    qrow = lax.broadcasted_iota(jnp.int32, (LQ, 1), 0).astype(jnp.float32)
    nt = (((1,), (1,)), ((), ()))
    nn = (((1,), (0,)), ((), ()))

    ones_row = jnp.ones((1, EMB), jnp.float32)
    row_acc = jnp.zeros((1, BB), jnp.float32)
    for i in range(BB):
        q = qe_ref[i]
        d = de_ref[i]
        dots = lax.dot_general(q.astype(jnp.bfloat16), d.astype(jnp.bfloat16),
                               nt, preferred_element_type=jnp.float32)
        rq = _refined_rsqrt(jnp.sum(q * q, axis=1, keepdims=True))
        dsq = d * d
        dsq_h = _b16(dsq)
        dsq_l = dsq - dsq_h
        dn2 = (lax.dot_general(ones_row, dsq_h, nt,
                               preferred_element_type=jnp.float32)
               + lax.dot_general(ones_row, dsq_l, nt,
                                 preferred_element_type=jnp.float32))
        inter = dots * rq * _refined_rsqrt(dn2)
        y = inter + jnp.float32(1.0)
        s = [None] * (NBINS + 1)
        for g in range(10):
            ks = [kk for kk in (3 * g + 1, 3 * g + 2, 3 * g + 3) if kk < NBINS]
            p = None
            for j, kk in enumerate(ks):
                t = jnp.where(y >= jnp.float32(_THRESH[kk - 1]),
                              jnp.float32(256.0 ** j), jnp.float32(0.0))
                p = t if p is None else p + t
            r = jnp.sum(p, axis=1, keepdims=True)
            c2 = jnp.floor(r * jnp.float32(1.0 / 65536.0))
            rem = r - c2 * jnp.float32(65536.0)
            c1 = jnp.floor(rem * jnp.float32(1.0 / 256.0))
            c0 = rem - c1 * jnp.float32(256.0)
            dec = (c0, c1, c2)
            for j, kk in enumerate(ks):
                s[kk] = dec[j]
        mask = (qrow < qlen_ref[i]).astype(jnp.float32)
        cols = []
        for k in range(NBINS):
            if k == 0:
                cols.append(jnp.float32(LD) - s[1])
            elif k == NBINS - 1:
                cols.append(s[NBINS - 1])
            else:
                cols.append(s[k] - s[k + 1])
        cols.append(jnp.zeros((LQ, 2), jnp.float32))
        hcnt = jnp.concatenate(cols, axis=1)
        lh = _b16(jnp.log1p(hcnt * mask))
        z1 = jnp.tanh(lax.dot_general(lh, w1t_ref[...], nn,
                                      preferred_element_type=jnp.float32)
                      + b1_ref[...])
        z2 = jnp.tanh(lax.dot_general(z1, w2t_ref[...], nn,
                                      preferred_element_type=jnp.float32)
                      + b2_ref[...])
        z3 = jnp.tanh(z2 * w3_ref[...] + b3_ref[...])
        glog = lax.dot_general(q, wgt_ref[...], nn,
                               preferred_element_type=jnp.float32) + bg_ref[...]
        gm = glog * col0 + (col0 - 1.0) * jnp.float32(1e9)
        m = jnp.max(gm)
        e = jnp.exp(gm - m)
        score = jnp.sum(z3 * e) / jnp.sum(e)
        row_acc = row_acc + jnp.where(iota_bb == i, score, 0.0)
    out_ref[...] = row_acc.reshape(1, 1, BB)


def _tc_call(qe, de, qlenf, w1t, b1p, w2t, b2p, w3p, b3p, wgt, bgp):
    grid = BATCH // BB
    full = lambda shape: pl.BlockSpec(shape, lambda i: (0,) * len(shape))
    out = pl.pallas_call(
        _tc_body,
        grid=(grid,),
        in_specs=[
            pl.BlockSpec((BB, LQ, EMB), lambda i: (i, 0, 0)),
            pl.BlockSpec((BB, LD, EMB), lambda i: (i, 0, 0)),
            pl.BlockSpec((BB, 1, 1), lambda i: (i, 0, 0)),
            full((32, 8)), full((1, 8)), full((8, 8)), full((1, 8)),
            full((1, 8)), full((1, 8)), full((EMB, 8)), full((1, 8)),
        ],
        out_specs=pl.BlockSpec((1, 1, BB), lambda i: (i, 0, 0)),
        out_shape=jax.ShapeDtypeStruct((grid, 1, BB), jnp.float32),
    )(qe, de, qlenf, w1t, b1p, w2t, b2p, w3p, b3p, wgt, bgp)
    return out.reshape(BATCH)


def kernel(query, query_len, document, table, W1, b1, W2, b2, W3, b3, Wg, bg):
    qidx = query.reshape(-1).astype(jnp.int32)
    didx = document.reshape(-1).astype(jnp.int32)
    q_emb, d_emb = _sc_gather(table, qidx, didx)
    qe = q_emb.reshape(BATCH, LQ, EMB)
    de = d_emb.reshape(BATCH, LD, EMB)
    qlenf = query_len.astype(jnp.float32).reshape(BATCH, 1, 1)
    f32 = jnp.float32
    r16 = lambda x: x.astype(jnp.bfloat16).astype(jnp.float32)
    w1t = jnp.zeros((32, 8), f32).at[:NBINS, :5].set(r16(W1.T))
    b1p = jnp.zeros((1, 8), f32).at[0, :5].set(b1)
    w2t = jnp.zeros((8, 8), f32).at[:5, 0].set(W2[0])
    b2p = jnp.zeros((1, 8), f32).at[0, 0].set(b2[0])
    w3p = jnp.zeros((1, 8), f32).at[0, 0].set(W3[0, 0])
    b3p = jnp.zeros((1, 8), f32).at[0, 0].set(b3[0])
    wgt = jnp.zeros((EMB, 8), f32).at[:, 0].set(Wg[0])
    bgp = jnp.zeros((1, 8), f32).at[0, 0].set(bg[0])
    return _tc_call(qe, de, qlenf, w1t, b1p, w2t, b2p, w3p, b3p, wgt, bgp)


# speedup vs baseline: 14.6086x; 2.1643x over previous
import functools

import jax
import jax.numpy as jnp
from jax import lax
from jax.experimental import pallas as pl
from jax.experimental.pallas import tpu as pltpu
from jax.experimental.pallas import tpu_sc as plsc

BATCH = 1024
LQ = 20
LD = 200
EMB = 128
NBINS = 30
BB = 8
CH = 128
DEPTH = 5


def _bin_thresholds():
    import numpy as np
    w = np.float32(2.0 / NBINS)
    out = []
    for k in range(1, NBINS):
        y = np.float32(np.float64(k) * np.float64(w))
        kf = np.float32(k)
        if np.float32(y / w) >= kf:
            while True:
                y2 = np.nextafter(y, np.float32(-np.inf), dtype=np.float32)
                if np.float32(y2 / w) >= kf:
                    y = y2
                else:
                    break
        else:
            while np.float32(y / w) < kf:
                y = np.nextafter(y, np.float32(np.inf), dtype=np.float32)
        out.append(float(y))
    return out


_THRESH = _bin_thresholds()


def _refined_rsqrt(x):
    x = jnp.maximum(x, jnp.float32(1e-16))
    r = lax.rsqrt(x)
    return r * (jnp.float32(1.5) - jnp.float32(0.5) * x * r * r)


def _b16(x):
    return x.astype(jnp.bfloat16).astype(jnp.float32)


def _sc_gather(table, qidx, didx):
    info = plsc.get_sparse_core_info()
    nc, ns = info.num_cores, info.num_subcores
    nw = nc * ns
    nq, nd = BATCH * LQ, BATCH * LD
    nch_q = nq // (nw * CH)
    nch_d = nd // (nw * CH)
    assert nq % (nw * CH) == 0 and nd % (nw * CH) == 0
    assert nch_q % DEPTH == 0 and nch_d % DEPTH == 0

    qidx3 = qidx.reshape(nw, nch_q, CH)
    didx3 = didx.reshape(nw, nch_d, CH)

    mesh = plsc.VectorSubcoreMesh(core_axis_name="c", subcore_axis_name="s")

    @functools.partial(
        pl.kernel,
        mesh=mesh,
        out_type=[
            jax.ShapeDtypeStruct((nq, EMB), jnp.float32),
            jax.ShapeDtypeStruct((nd, EMB), jnp.float32),
        ],
        scratch_types=[
            pltpu.VMEM((nch_q, CH), jnp.int32),
            pltpu.VMEM((nch_d, CH), jnp.int32),
            pltpu.VMEM((DEPTH, CH, EMB), jnp.float32),
        ]
        + [pltpu.SemaphoreType.DMA] * (2 * DEPTH),
    )
    def gather_kernel(tbl, qi3, di3, qout, dout, qidx_v, didx_v, rows_v, *sems):
        gsem = sems[:DEPTH]
        osem = sems[DEPTH:]
        wid = lax.axis_index("s") * nc + lax.axis_index("c")
        pltpu.sync_copy(di3.at[wid], didx_v)
        pltpu.sync_copy(qi3.at[wid], qidx_v)

        def phase(idx_v, out_hbm, nch):
            base = wid * (nch * CH)

            def round_body(r, carry):
                for b in range(DEPTH):
                    c = r * DEPTH + b
                    pltpu.make_async_copy(
                        tbl.at[idx_v.at[c]], rows_v.at[b], gsem[b]
                    ).start()
                for b in range(DEPTH):
                    c = r * DEPTH + b
                    pltpu.make_async_copy(
                        tbl.at[idx_v.at[c]], rows_v.at[b], gsem[b]
                    ).wait()
                    pltpu.make_async_copy(
                        rows_v.at[b],
                        out_hbm.at[pl.ds(base + c * CH, CH)],
                        osem[b],
                    ).start()
                for b in range(DEPTH):
                    c = r * DEPTH + b
                    pltpu.make_async_copy(
                        rows_v.at[b],
                        out_hbm.at[pl.ds(base + c * CH, CH)],
                        osem[b],
                    ).wait()
                return carry

            lax.fori_loop(0, nch // DEPTH, round_body, 0)

        phase(didx_v, dout, nch_d)
        phase(qidx_v, qout, nch_q)

    return gather_kernel(table, qidx3, didx3)


NL = BB * LQ


def _tc_body(qe_ref, de_ref, qlen_ref, w1_ref, b1_ref, w2_ref, b2_ref,
             w3_ref, b3_ref, wg_ref, bg_ref, out_ref, y_sc):
    f32 = jnp.float32
    nt = (((1,), (1,)), ((), ()))
    ones_row = jnp.ones((1, EMB), f32)

    for i in range(BB):
        q = qe_ref[i]
        d = de_ref[i]
        dotsT = lax.dot_general(d.astype(jnp.bfloat16), q.astype(jnp.bfloat16),
                                nt, preferred_element_type=f32)
        qsq = q * q
        qsq_h = _b16(qsq)
        qsq_l = qsq - qsq_h
        qn2 = (lax.dot_general(ones_row, qsq_h, nt, preferred_element_type=f32)
               + lax.dot_general(ones_row, qsq_l, nt,
                                 preferred_element_type=f32))
        rdc = _refined_rsqrt(jnp.sum(d * d, axis=1, keepdims=True))
        yT = dotsT * rdc * _refined_rsqrt(qn2) + f32(1.0)
        y_sc[:, i * LQ:(i + 1) * LQ] = yT

    Y = y_sc[...]
    s = [None] * (NBINS + 1)
    for g in range(10):
        ks = [kk for kk in (3 * g + 1, 3 * g + 2, 3 * g + 3) if kk < NBINS]
        p = None
        for j, kk in enumerate(ks):
            t = jnp.where(Y >= f32(_THRESH[kk - 1]), f32(256.0 ** j), f32(0.0))
            p = t if p is None else p + t
        r = jnp.sum(p, axis=0, keepdims=True)
        c2 = jnp.floor(r * f32(1.0 / 65536.0))
        rem = r - c2 * f32(65536.0)
        c1 = jnp.floor(rem * f32(1.0 / 256.0))
        dec = (rem - c1 * f32(256.0), c1, c2)
        for j, kk in enumerate(ks):
            s[kk] = dec[j]

    il = lax.broadcasted_iota(jnp.int32, (1, NL), 1).astype(f32)
    bi = jnp.floor(il * f32(0.05))
    qpos = il - f32(LQ) * bi
    seg = (bi == lax.broadcasted_iota(jnp.int32, (BB, NL), 0).astype(f32))
    seg = seg.astype(f32)
    qlen8 = qlen_ref[...].reshape(1, BB)
    qlen_row = lax.dot_general(qlen8, seg, (((1,), (0,)), ((), ())),
                               preferred_element_type=f32)
    mask = (qpos < qlen_row).astype(f32)

    acc = jnp.zeros((8, NL), f32)
    for k in range(NBINS):
        if k == 0:
            cnt = f32(LD) - s[1]
        elif k == NBINS - 1:
            cnt = s[NBINS - 1]
        else:
            cnt = s[k] - s[k + 1]
        lh = _b16(jnp.log1p(cnt * mask))
        acc = acc + lh * w1_ref[:, k:k + 1]
    z1 = jnp.tanh(acc + b1_ref[...])
    z2 = jnp.tanh(jnp.sum(_b16(z1) * w2_ref[...], axis=0, keepdims=True)
                  + b2_ref[...])
    z3 = jnp.tanh(z2 * w3_ref[...] + b3_ref[...])

    qall = qe_ref[...].reshape(NL, EMB)
    glog = lax.dot_general(wg_ref[...], qall, nt,
                           preferred_element_type=f32) + bg_ref[...]
    e = jnp.exp(glog)
    z3e = z3 * e
    eh = _b16(e)
    zh = _b16(z3e)
    den = (lax.dot_general(eh, seg, nt, preferred_element_type=f32)
           + lax.dot_general(e - eh, seg, nt, preferred_element_type=f32))
    num = (lax.dot_general(zh, seg, nt, preferred_element_type=f32)
           + lax.dot_general(z3e - zh, seg, nt, preferred_element_type=f32))
    out_ref[...] = (num / den).reshape(1, 1, BB)


def _tc_call(qe, de, qlenf, w1p, b1c, w2c, b2s, w3s, b3s, wg, bgs):
    grid = BATCH // BB
    full = lambda shape: pl.BlockSpec(shape, lambda i: (0,) * len(shape))
    out = pl.pallas_call(
        _tc_body,
        grid=(grid,),
        in_specs=[
            pl.BlockSpec((BB, LQ, EMB), lambda i: (i, 0, 0)),
            pl.BlockSpec((BB, LD, EMB), lambda i: (i, 0, 0)),
            pl.BlockSpec((1, 1, BB), lambda i: (i, 0, 0)),
            full((8, 32)), full((8, 1)), full((8, 1)), full((1, 1)),
            full((1, 1)), full((1, 1)), full((1, EMB)), full((1, 1)),
        ],
        out_specs=pl.BlockSpec((1, 1, BB), lambda i: (i, 0, 0)),
        out_shape=jax.ShapeDtypeStruct((grid, 1, BB), jnp.float32),
        scratch_shapes=[pltpu.VMEM((LD, NL), jnp.float32)],
    )(qe, de, qlenf, w1p, b1c, w2c, b2s, w3s, b3s, wg, bgs)
    return out.reshape(BATCH)


def kernel(query, query_len, document, table, W1, b1, W2, b2, W3, b3, Wg, bg):
    qidx = query.reshape(-1).astype(jnp.int32)
    didx = document.reshape(-1).astype(jnp.int32)
    q_emb, d_emb = _sc_gather(table, qidx, didx)
    qe = q_emb.reshape(BATCH, LQ, EMB)
    de = d_emb.reshape(BATCH, LD, EMB)
    qlenf = query_len.astype(jnp.float32).reshape(BATCH // BB, 1, BB)
    f32 = jnp.float32
    r16 = lambda x: x.astype(jnp.bfloat16).astype(jnp.float32)
    w1p = jnp.zeros((8, 32), f32).at[:5, :NBINS].set(r16(W1))
    b1c = jnp.zeros((8, 1), f32).at[:5, 0].set(b1)
    w2c = jnp.zeros((8, 1), f32).at[:5, 0].set(r16(W2[0]))
    b2s = b2.reshape(1, 1).astype(f32)
    w3s = W3.reshape(1, 1).astype(f32)
    b3s = b3.reshape(1, 1).astype(f32)
    wg = Wg.reshape(1, EMB).astype(f32)
    bgs = bg.reshape(1, 1).astype(f32)
    return _tc_call(qe, de, qlenf, w1p, b1c, w2c, b2s, w3s, b3s, wg, bgs)


# speedup vs baseline: 16.4354x; 1.1251x over previous
import functools

import jax
import jax.numpy as jnp
from jax import lax
from jax.experimental import pallas as pl
from jax.experimental.pallas import tpu as pltpu
from jax.experimental.pallas import tpu_sc as plsc

BATCH = 1024
LQ = 20
LD = 200
EMB = 128
NBINS = 30
BB = 16
CH = 128
DEPTH = 5


def _bin_thresholds():
    import numpy as np
    w = np.float32(2.0 / NBINS)
    out = []
    for k in range(1, NBINS):
        y = np.float32(np.float64(k) * np.float64(w))
        kf = np.float32(k)
        if np.float32(y / w) >= kf:
            while True:
                y2 = np.nextafter(y, np.float32(-np.inf), dtype=np.float32)
                if np.float32(y2 / w) >= kf:
                    y = y2
                else:
                    break
        else:
            while np.float32(y / w) < kf:
                y = np.nextafter(y, np.float32(np.inf), dtype=np.float32)
        out.append(float(y))
    return out


_THRESH = _bin_thresholds()


def _refined_rsqrt(x):
    x = jnp.maximum(x, jnp.float32(1e-16))
    r = lax.rsqrt(x)
    return r * (jnp.float32(1.5) - jnp.float32(0.5) * x * r * r)


def _b16(x):
    return x.astype(jnp.bfloat16).astype(jnp.float32)


def _sc_gather(table, qidx, didx):
    info = plsc.get_sparse_core_info()
    nc, ns = info.num_cores, info.num_subcores
    nw = nc * ns
    nq, nd = BATCH * LQ, BATCH * LD
    nch_q = nq // (nw * CH)
    nch_d = nd // (nw * CH)
    assert nq % (nw * CH) == 0 and nd % (nw * CH) == 0
    assert nch_q % DEPTH == 0 and nch_d % DEPTH == 0

    qidx3 = qidx.reshape(nw, nch_q, CH)
    didx3 = didx.reshape(nw, nch_d, CH)

    mesh = plsc.VectorSubcoreMesh(core_axis_name="c", subcore_axis_name="s")

    @functools.partial(
        pl.kernel,
        mesh=mesh,
        out_type=[
            jax.ShapeDtypeStruct((nq, EMB), jnp.float32),
            jax.ShapeDtypeStruct((nd, EMB), jnp.float32),
        ],
        scratch_types=[
            pltpu.VMEM((nch_q, CH), jnp.int32),
            pltpu.VMEM((nch_d, CH), jnp.int32),
            pltpu.VMEM((DEPTH, CH, EMB), jnp.float32),
        ]
        + [pltpu.SemaphoreType.DMA] * (2 * DEPTH),
    )
    def gather_kernel(tbl, qi3, di3, qout, dout, qidx_v, didx_v, rows_v, *sems):
        gsem = sems[:DEPTH]
        osem = sems[DEPTH:]
        wid = lax.axis_index("s") * nc + lax.axis_index("c")
        pltpu.sync_copy(di3.at[wid], didx_v)
        pltpu.sync_copy(qi3.at[wid], qidx_v)

        def phase(idx_v, out_hbm, nch):
            base = wid * (nch * CH)

            def round_body(r, carry):
                for b in range(DEPTH):
                    c = r * DEPTH + b
                    pltpu.make_async_copy(
                        tbl.at[idx_v.at[c]], rows_v.at[b], gsem[b]
                    ).start()
                for b in range(DEPTH):
                    c = r * DEPTH + b
                    pltpu.make_async_copy(
                        tbl.at[idx_v.at[c]], rows_v.at[b], gsem[b]
                    ).wait()
                    pltpu.make_async_copy(
                        rows_v.at[b],
                        out_hbm.at[pl.ds(base + c * CH, CH)],
                        osem[b],
                    ).start()
                for b in range(DEPTH):
                    c = r * DEPTH + b
                    pltpu.make_async_copy(
                        rows_v.at[b],
                        out_hbm.at[pl.ds(base + c * CH, CH)],
                        osem[b],
                    ).wait()
                return carry

            lax.fori_loop(0, nch // DEPTH, round_body, 0)

        phase(didx_v, dout, nch_d)
        phase(qidx_v, qout, nch_q)

    return gather_kernel(table, qidx3, didx3)


NL = BB * LQ


def _tc_body(qe_ref, de_ref, qlen_ref, w1_ref, b1_ref, w2_ref, b2_ref,
             w3_ref, b3_ref, wg_ref, bg_ref, out_ref, y_sc):
    f32 = jnp.float32
    nt = (((1,), (1,)), ((), ()))
    ones_row = jnp.ones((1, EMB), f32)

    for i in range(BB):
        q = qe_ref[i]
        d = de_ref[i]
        dotsT = lax.dot_general(d.astype(jnp.bfloat16), q.astype(jnp.bfloat16),
                                nt, preferred_element_type=f32)
        qsq = q * q
        qsq_h = _b16(qsq)
        qsq_l = qsq - qsq_h
        qn2 = (lax.dot_general(ones_row, qsq_h, nt, preferred_element_type=f32)
               + lax.dot_general(ones_row, qsq_l, nt,
                                 preferred_element_type=f32))
        rdc = _refined_rsqrt(jnp.sum(d * d, axis=1, keepdims=True))
        yT = dotsT * rdc * _refined_rsqrt(qn2) + f32(1.0)
        y_sc[:, i * LQ:(i + 1) * LQ] = yT

    Y = y_sc[...]
    s = [None] * (NBINS + 1)
    for g in range(10):
        ks = [kk for kk in (3 * g + 1, 3 * g + 2, 3 * g + 3) if kk < NBINS]
        p = None
        for j, kk in enumerate(ks):
            t = jnp.where(Y >= f32(_THRESH[kk - 1]), f32(256.0 ** j), f32(0.0))
            p = t if p is None else p + t
        r = jnp.sum(p, axis=0, keepdims=True)
        c2 = jnp.floor(r * f32(1.0 / 65536.0))
        rem = r - c2 * f32(65536.0)
        c1 = jnp.floor(rem * f32(1.0 / 256.0))
        dec = (rem - c1 * f32(256.0), c1, c2)
        for j, kk in enumerate(ks):
            s[kk] = dec[j]

    il = lax.broadcasted_iota(jnp.int32, (1, NL), 1).astype(f32)
    bi = jnp.floor(il * f32(0.05))
    qpos = il - f32(LQ) * bi
    seg = (bi == lax.broadcasted_iota(jnp.int32, (BB, NL), 0).astype(f32))
    seg = seg.astype(f32)
    qlen8 = qlen_ref[...].reshape(1, BB)
    qlen_row = lax.dot_general(qlen8, seg, (((1,), (0,)), ((), ())),
                               preferred_element_type=f32)
    mask = (qpos < qlen_row).astype(f32)

    acc = jnp.zeros((8, NL), f32)
    for k in range(NBINS):
        if k == 0:
            cnt = f32(LD) - s[1]
        elif k == NBINS - 1:
            cnt = s[NBINS - 1]
        else:
            cnt = s[k] - s[k + 1]
        lh = _b16(jnp.log1p(cnt * mask))
        acc = acc + lh * w1_ref[:, k:k + 1]
    z1 = jnp.tanh(acc + b1_ref[...])
    z2 = jnp.tanh(jnp.sum(_b16(z1) * w2_ref[...], axis=0, keepdims=True)
                  + b2_ref[...])
    z3 = jnp.tanh(z2 * w3_ref[...] + b3_ref[...])

    qall = qe_ref[...].reshape(NL, EMB)
    glog = lax.dot_general(wg_ref[...], qall, nt,
                           preferred_element_type=f32) + bg_ref[...]
    e = jnp.exp(glog)
    z3e = z3 * e
    eh = _b16(e)
    zh = _b16(z3e)
    den = (lax.dot_general(eh, seg, nt, preferred_element_type=f32)
           + lax.dot_general(e - eh, seg, nt, preferred_element_type=f32))
    num = (lax.dot_general(zh, seg, nt, preferred_element_type=f32)
           + lax.dot_general(z3e - zh, seg, nt, preferred_element_type=f32))
    out_ref[...] = (num / den).reshape(1, 1, BB)


def _tc_call(qe, de, qlenf, w1p, b1c, w2c, b2s, w3s, b3s, wg, bgs):
    grid = BATCH // BB
    full = lambda shape: pl.BlockSpec(shape, lambda i: (0,) * len(shape))
    out = pl.pallas_call(
        _tc_body,
        grid=(grid,),
        in_specs=[
            pl.BlockSpec((BB, LQ, EMB), lambda i: (i, 0, 0)),
            pl.BlockSpec((BB, LD, EMB), lambda i: (i, 0, 0)),
            pl.BlockSpec((1, 1, BB), lambda i: (i, 0, 0)),
            full((8, 32)), full((8, 1)), full((8, 1)), full((1, 1)),
            full((1, 1)), full((1, 1)), full((1, EMB)), full((1, 1)),
        ],
        out_specs=pl.BlockSpec((1, 1, BB), lambda i: (i, 0, 0)),
        out_shape=jax.ShapeDtypeStruct((grid, 1, BB), jnp.float32),
        scratch_shapes=[pltpu.VMEM((LD, NL), jnp.float32)],
    )(qe, de, qlenf, w1p, b1c, w2c, b2s, w3s, b3s, wg, bgs)
    return out.reshape(BATCH)


def kernel(query, query_len, document, table, W1, b1, W2, b2, W3, b3, Wg, bg):
    qidx = query.reshape(-1).astype(jnp.int32)
    didx = document.reshape(-1).astype(jnp.int32)
    q_emb, d_emb = _sc_gather(table, qidx, didx)
    qe = q_emb.reshape(BATCH, LQ, EMB)
    de = d_emb.reshape(BATCH, LD, EMB)
    qlenf = query_len.astype(jnp.float32).reshape(BATCH // BB, 1, BB)
    f32 = jnp.float32
    r16 = lambda x: x.astype(jnp.bfloat16).astype(jnp.float32)
    w1p = jnp.zeros((8, 32), f32).at[:5, :NBINS].set(r16(W1))
    b1c = jnp.zeros((8, 1), f32).at[:5, 0].set(b1)
    w2c = jnp.zeros((8, 1), f32).at[:5, 0].set(r16(W2[0]))
    b2s = b2.reshape(1, 1).astype(f32)
    w3s = W3.reshape(1, 1).astype(f32)
    b3s = b3.reshape(1, 1).astype(f32)
    wg = Wg.reshape(1, EMB).astype(f32)
    bgs = bg.reshape(1, 1).astype(f32)
    return _tc_call(qe, de, qlenf, w1p, b1c, w2c, b2s, w3s, b3s, wg, bgs)


# speedup vs baseline: 17.9588x; 1.0927x over previous
import functools

import jax
import jax.numpy as jnp
from jax import lax
from jax.experimental import pallas as pl
from jax.experimental.pallas import tpu as pltpu
from jax.experimental.pallas import tpu_sc as plsc

BATCH = 1024
LQ = 20
LD = 200
EMB = 128
NBINS = 30
BB = 32
CH = 128
DEPTH = 5


def _bin_thresholds():
    import numpy as np
    w = np.float32(2.0 / NBINS)
    out = []
    for k in range(1, NBINS):
        y = np.float32(np.float64(k) * np.float64(w))
        kf = np.float32(k)
        if np.float32(y / w) >= kf:
            while True:
                y2 = np.nextafter(y, np.float32(-np.inf), dtype=np.float32)
                if np.float32(y2 / w) >= kf:
                    y = y2
                else:
                    break
        else:
            while np.float32(y / w) < kf:
                y = np.nextafter(y, np.float32(np.inf), dtype=np.float32)
        out.append(float(y))
    return out


_THRESH = _bin_thresholds()


def _refined_rsqrt(x):
    x = jnp.maximum(x, jnp.float32(1e-16))
    r = lax.rsqrt(x)
    return r * (jnp.float32(1.5) - jnp.float32(0.5) * x * r * r)


def _b16(x):
    return x.astype(jnp.bfloat16).astype(jnp.float32)


def _sc_gather(table, qidx, didx):
    info = plsc.get_sparse_core_info()
    nc, ns = info.num_cores, info.num_subcores
    nw = nc * ns
    nq, nd = BATCH * LQ, BATCH * LD
    nch_q = nq // (nw * CH)
    nch_d = nd // (nw * CH)
    assert nq % (nw * CH) == 0 and nd % (nw * CH) == 0
    assert nch_q % DEPTH == 0 and nch_d % DEPTH == 0

    qidx3 = qidx.reshape(nw, nch_q, CH)
    didx3 = didx.reshape(nw, nch_d, CH)

    mesh = plsc.VectorSubcoreMesh(core_axis_name="c", subcore_axis_name="s")

    @functools.partial(
        pl.kernel,
        mesh=mesh,
        out_type=[
            jax.ShapeDtypeStruct((nq, EMB), jnp.float32),
            jax.ShapeDtypeStruct((nd, EMB), jnp.float32),
        ],
        scratch_types=[
            pltpu.VMEM((nch_q, CH), jnp.int32),
            pltpu.VMEM((nch_d, CH), jnp.int32),
            pltpu.VMEM((DEPTH, CH, EMB), jnp.float32),
        ]
        + [pltpu.SemaphoreType.DMA] * (2 * DEPTH),
    )
    def gather_kernel(tbl, qi3, di3, qout, dout, qidx_v, didx_v, rows_v, *sems):
        gsem = sems[:DEPTH]
        osem = sems[DEPTH:]
        wid = lax.axis_index("s") * nc + lax.axis_index("c")
        pltpu.sync_copy(di3.at[wid], didx_v)
        pltpu.sync_copy(qi3.at[wid], qidx_v)

        def phase(idx_v, out_hbm, nch):
            base = wid * (nch * CH)

            def round_body(r, carry):
                for b in range(DEPTH):
                    c = r * DEPTH + b
                    pltpu.make_async_copy(
                        tbl.at[idx_v.at[c]], rows_v.at[b], gsem[b]
                    ).start()
                for b in range(DEPTH):
                    c = r * DEPTH + b
                    pltpu.make_async_copy(
                        tbl.at[idx_v.at[c]], rows_v.at[b], gsem[b]
                    ).wait()
                    pltpu.make_async_copy(
                        rows_v.at[b],
                        out_hbm.at[pl.ds(base + c * CH, CH)],
                        osem[b],
                    ).start()
                for b in range(DEPTH):
                    c = r * DEPTH + b
                    pltpu.make_async_copy(
                        rows_v.at[b],
                        out_hbm.at[pl.ds(base + c * CH, CH)],
                        osem[b],
                    ).wait()
                return carry

            lax.fori_loop(0, nch // DEPTH, round_body, 0)

        phase(didx_v, dout, nch_d)
        phase(qidx_v, qout, nch_q)

    return gather_kernel(table, qidx3, didx3)


NL = BB * LQ


def _tc_body(qe_ref, de_ref, qlen_ref, w1_ref, b1_ref, w2_ref, b2_ref,
             w3_ref, b3_ref, wg_ref, bg_ref, out_ref, y_sc):
    f32 = jnp.float32
    nt = (((1,), (1,)), ((), ()))
    ones_row = jnp.ones((1, EMB), f32)

    for i in range(BB):
        q = qe_ref[i]
        d = de_ref[i]
        dotsT = lax.dot_general(d.astype(jnp.bfloat16), q.astype(jnp.bfloat16),
                                nt, preferred_element_type=f32)
        qsq = q * q
        qsq_h = _b16(qsq)
        qsq_l = qsq - qsq_h
        qn2 = (lax.dot_general(ones_row, qsq_h, nt, preferred_element_type=f32)
               + lax.dot_general(ones_row, qsq_l, nt,
                                 preferred_element_type=f32))
        rdc = _refined_rsqrt(jnp.sum(d * d, axis=1, keepdims=True))
        yT = dotsT * rdc * _refined_rsqrt(qn2) + f32(1.0)
        y_sc[:, i * LQ:(i + 1) * LQ] = yT

    Y = y_sc[...]
    s = [None] * (NBINS + 1)
    for g in range(10):
        ks = [kk for kk in (3 * g + 1, 3 * g + 2, 3 * g + 3) if kk < NBINS]
        p = None
        for j, kk in enumerate(ks):
            t = jnp.where(Y >= f32(_THRESH[kk - 1]), f32(256.0 ** j), f32(0.0))
            p = t if p is None else p + t
        r = jnp.sum(p, axis=0, keepdims=True)
        c2 = jnp.floor(r * f32(1.0 / 65536.0))
        rem = r - c2 * f32(65536.0)
        c1 = jnp.floor(rem * f32(1.0 / 256.0))
        dec = (rem - c1 * f32(256.0), c1, c2)
        for j, kk in enumerate(ks):
            s[kk] = dec[j]

    il = lax.broadcasted_iota(jnp.int32, (1, NL), 1).astype(f32)
    bi = jnp.floor(il * f32(0.05))
    qpos = il - f32(LQ) * bi
    seg = (bi == lax.broadcasted_iota(jnp.int32, (BB, NL), 0).astype(f32))
    seg = seg.astype(f32)
    qlen8 = qlen_ref[...].reshape(1, BB)
    qlen_row = lax.dot_general(qlen8, seg, (((1,), (0,)), ((), ())),
                               preferred_element_type=f32)
    mask = (qpos < qlen_row).astype(f32)

    acc = jnp.zeros((8, NL), f32)
    for k in range(NBINS):
        if k == 0:
            cnt = f32(LD) - s[1]
        elif k == NBINS - 1:
            cnt = s[NBINS - 1]
        else:
            cnt = s[k] - s[k + 1]
        lh = _b16(jnp.log1p(cnt * mask))
        acc = acc + lh * w1_ref[:, k:k + 1]
    z1 = jnp.tanh(acc + b1_ref[...])
    z2 = jnp.tanh(jnp.sum(_b16(z1) * w2_ref[...], axis=0, keepdims=True)
                  + b2_ref[...])
    z3 = jnp.tanh(z2 * w3_ref[...] + b3_ref[...])

    qall = qe_ref[...].reshape(NL, EMB)
    glog = lax.dot_general(wg_ref[...], qall, nt,
                           preferred_element_type=f32) + bg_ref[...]
    e = jnp.exp(glog)
    z3e = z3 * e
    eh = _b16(e)
    zh = _b16(z3e)
    den = (lax.dot_general(eh, seg, nt, preferred_element_type=f32)
           + lax.dot_general(e - eh, seg, nt, preferred_element_type=f32))
    num = (lax.dot_general(zh, seg, nt, preferred_element_type=f32)
           + lax.dot_general(z3e - zh, seg, nt, preferred_element_type=f32))
    out_ref[...] = (num / den).reshape(1, 1, BB)


def _tc_call(qe, de, qlenf, w1p, b1c, w2c, b2s, w3s, b3s, wg, bgs):
    grid = BATCH // BB
    full = lambda shape: pl.BlockSpec(shape, lambda i: (0,) * len(shape))
    out = pl.pallas_call(
        _tc_body,
        grid=(grid,),
        in_specs=[
            pl.BlockSpec((BB, LQ, EMB), lambda i: (i, 0, 0)),
            pl.BlockSpec((BB, LD, EMB), lambda i: (i, 0, 0)),
            pl.BlockSpec((1, 1, BB), lambda i: (i, 0, 0)),
            full((8, 32)), full((8, 1)), full((8, 1)), full((1, 1)),
            full((1, 1)), full((1, 1)), full((1, EMB)), full((1, 1)),
        ],
        out_specs=pl.BlockSpec((1, 1, BB), lambda i: (i, 0, 0)),
        out_shape=jax.ShapeDtypeStruct((grid, 1, BB), jnp.float32),
        scratch_shapes=[pltpu.VMEM((LD, NL), jnp.float32)],
    )(qe, de, qlenf, w1p, b1c, w2c, b2s, w3s, b3s, wg, bgs)
    return out.reshape(BATCH)


def kernel(query, query_len, document, table, W1, b1, W2, b2, W3, b3, Wg, bg):
    qidx = query.reshape(-1).astype(jnp.int32)
    didx = document.reshape(-1).astype(jnp.int32)
    q_emb, d_emb = _sc_gather(table, qidx, didx)
    qe = q_emb.reshape(BATCH, LQ, EMB)
    de = d_emb.reshape(BATCH, LD, EMB)
    qlenf = query_len.astype(jnp.float32).reshape(BATCH // BB, 1, BB)
    f32 = jnp.float32
    r16 = lambda x: x.astype(jnp.bfloat16).astype(jnp.float32)
    w1p = jnp.zeros((8, 32), f32).at[:5, :NBINS].set(r16(W1))
    b1c = jnp.zeros((8, 1), f32).at[:5, 0].set(b1)
    w2c = jnp.zeros((8, 1), f32).at[:5, 0].set(r16(W2[0]))
    b2s = b2.reshape(1, 1).astype(f32)
    w3s = W3.reshape(1, 1).astype(f32)
    b3s = b3.reshape(1, 1).astype(f32)
    wg = Wg.reshape(1, EMB).astype(f32)
    bgs = bg.reshape(1, 1).astype(f32)
    return _tc_call(qe, de, qlenf, w1p, b1c, w2c, b2s, w3s, b3s, wg, bgs)
